# Initial kernel scaffold; baseline (speedup 1.0000x reference)
import jax
import jax.numpy as jnp
from jax.experimental import pallas as pl


def kernel(x, upsamples):
    raise NotImplementedError("write your pallas kernel here")


# speedup vs baseline: 1.9186x; 1.9186x over previous
import functools

import jax
import jax.numpy as jnp
from jax import lax
from jax.experimental import pallas as pl
from jax.experimental.pallas import tpu as pltpu
from jax.experimental.pallas import tpu_sc as plsc

_CHUNK = 128


def kernel(x, upsamples):
    n_out = upsamples.shape[0]
    d = x.shape[1]
    info = plsc.get_sparse_core_info()
    nw = info.num_cores * info.num_subcores
    n_full = n_out // _CHUNK
    rem = n_out - n_full * _CHUNK
    assert rem % 8 == 0
    n_pad = (n_full + (1 if rem else 0)) * _CHUNK
    iters = -(-n_full // nw)

    inds = upsamples[:, 0].astype(jnp.int32)
    inds = jnp.pad(inds, (0, n_pad - n_out))
    mesh = plsc.VectorSubcoreMesh(core_axis_name="c", subcore_axis_name="s")

    @functools.partial(
        pl.kernel,
        out_type=jax.ShapeDtypeStruct((n_out, d), jnp.float32),
        mesh=mesh,
        scratch_types=[
            pltpu.VMEM((_CHUNK,), jnp.int32),
            pltpu.VMEM((_CHUNK, d), jnp.float32),
            pltpu.SemaphoreType.DMA,
        ],
    )
    def gather_kernel(x_hbm, inds_hbm, out_hbm, idx_v, rows_v, sem):
        wid = lax.axis_index("s") * info.num_cores + lax.axis_index("c")

        def body(k, carry):
            ch = wid + k * nw

            @pl.when(ch < n_full)
            def _():
                off = pl.multiple_of(ch * _CHUNK, _CHUNK)
                pltpu.sync_copy(inds_hbm.at[pl.ds(off, _CHUNK)], idx_v)
                pltpu.async_copy(x_hbm.at[idx_v], rows_v, sem).wait()
                pltpu.sync_copy(rows_v, out_hbm.at[pl.ds(off, _CHUNK)])

            return carry

        lax.fori_loop(0, iters, body, 0)

        if rem:

            @pl.when(wid == nw - 1)
            def _():
                pltpu.sync_copy(inds_hbm.at[pl.ds(n_full * _CHUNK, _CHUNK)], idx_v)
                pltpu.async_copy(x_hbm.at[idx_v], rows_v, sem).wait()
                pltpu.sync_copy(
                    rows_v.at[pl.ds(0, rem)],
                    out_hbm.at[pl.ds(n_full * _CHUNK, rem)],
                )

    return gather_kernel(x, inds)


# speedup vs baseline: 2.1691x; 1.1305x over previous
import functools

import jax
import jax.numpy as jnp
from jax import lax
from jax.experimental import pallas as pl
from jax.experimental.pallas import tpu as pltpu
from jax.experimental.pallas import tpu_sc as plsc

_CHUNK = 120


def kernel(x, upsamples):
    n_out = upsamples.shape[0]
    d = x.shape[1]
    info = plsc.get_sparse_core_info()
    nw = info.num_cores * info.num_subcores
    n_full = n_out // _CHUNK
    rem = n_out - n_full * _CHUNK
    assert rem % 8 == 0 and _CHUNK % 8 == 0
    per_w = n_full // nw
    extra = n_full - per_w * nw
    assert extra <= 1
    kmax = per_w + 1
    pre = kmax * _CHUNK
    n_pad = (n_full + (1 if rem else 0)) * _CHUNK

    inds = upsamples[:, 0].astype(jnp.int32)
    inds = jnp.pad(inds, (0, n_pad - n_out))
    mesh = plsc.VectorSubcoreMesh(core_axis_name="c", subcore_axis_name="s")

    @functools.partial(
        pl.kernel,
        out_type=jax.ShapeDtypeStruct((n_out, d), jnp.float32),
        mesh=mesh,
        scratch_types=[
            pltpu.VMEM((pre,), jnp.int32),
            pltpu.VMEM((_CHUNK, d), jnp.float32),
            pltpu.VMEM((_CHUNK, d), jnp.float32),
            pltpu.SemaphoreType.DMA,
            pltpu.SemaphoreType.DMA,
            pltpu.SemaphoreType.DMA,
            pltpu.SemaphoreType.DMA,
        ],
    )
    def gather_kernel(x_hbm, inds_hbm, out_hbm, idx_all, rows0, rows1,
                      gs0, gs1, ws0, ws1):
        wid = lax.axis_index("s") * info.num_cores + lax.axis_index("c")
        base = wid * per_w + jnp.minimum(wid, extra)
        n_my = jnp.where(wid < extra, per_w + 1, per_w)
        rows = (rows0, rows1)
        gs = (gs0, gs1)
        ws = (ws0, ws1)

        pltpu.sync_copy(
            inds_hbm.at[pl.ds(pl.multiple_of(base * _CHUNK, 8), pre)],
            idx_all,
        )

        def start_gather(k, b):
            idx_ref = idx_all.at[pl.ds(pl.multiple_of(k * _CHUNK, 8), _CHUNK)]
            pltpu.async_copy(x_hbm.at[idx_ref], rows[b], gs[b])

        def start_wb(k, b):
            off = pl.multiple_of((base + k) * _CHUNK, 8)
            pltpu.async_copy(rows[b], out_hbm.at[pl.ds(off, _CHUNK)], ws[b])

        def drain_g(b):
            pltpu.make_async_copy(
                x_hbm.at[pl.ds(0, _CHUNK)], rows[b], gs[b]).wait()

        def drain_w(b):
            pltpu.make_async_copy(
                rows[b], out_hbm.at[pl.ds(0, _CHUNK)], ws[b]).wait()

        start_gather(0, 0)

        def pair(pk, carry):
            for j in (1, 2):
                k = 2 * pk + j
                b = j & 1

                @pl.when((k >= 2) & (k < n_my))
                def _():
                    drain_w(b)

                @pl.when(k < n_my)
                def _():
                    start_gather(k, b)

                @pl.when(k - 1 < n_my)
                def _():
                    drain_g(1 - b)
                    start_wb(k - 1, 1 - b)

            return carry

        lax.fori_loop(0, (kmax + 2) // 2, pair, 0)
        drain_w(0)
        drain_w(1)

        if rem:

            @pl.when(wid == nw - 1)
            def _():
                start_gather(per_w, 0)
                drain_g(0)
                pltpu.sync_copy(
                    rows0.at[pl.ds(0, rem)],
                    out_hbm.at[pl.ds(n_full * _CHUNK, rem)],
                )

    return gather_kernel(x, inds)


# speedup vs baseline: 2.2123x; 1.0200x over previous
import functools

import jax
import jax.numpy as jnp
from jax import lax
from jax.experimental import pallas as pl
from jax.experimental.pallas import tpu as pltpu
from jax.experimental.pallas import tpu_sc as plsc

_CHUNK = 80
_NBUF = 3


def kernel(x, upsamples):
    n_out = upsamples.shape[0]
    d = x.shape[1]
    info = plsc.get_sparse_core_info()
    nw = info.num_cores * info.num_subcores
    assert n_out % _CHUNK == 0 and _CHUNK % 8 == 0
    n_chunks = n_out // _CHUNK
    per_w = n_chunks // nw
    extra = n_chunks - per_w * nw
    kmax = per_w + (1 if extra else 0)
    pre = kmax * _CHUNK
    pad_to = ((nw - 1) * per_w + min(nw - 1, extra) + kmax) * _CHUNK
    pad_to = max(pad_to, n_out)

    inds = upsamples[:, 0].astype(jnp.int32)
    inds = jnp.pad(inds, (0, pad_to - n_out))
    mesh = plsc.VectorSubcoreMesh(core_axis_name="c", subcore_axis_name="s")

    @functools.partial(
        pl.kernel,
        out_type=jax.ShapeDtypeStruct((n_out, d), jnp.float32),
        mesh=mesh,
        scratch_types=(
            [pltpu.VMEM((pre,), jnp.int32)]
            + [pltpu.VMEM((_CHUNK, d), jnp.float32)] * _NBUF
            + [pltpu.SemaphoreType.DMA] * (2 * _NBUF)
        ),
    )
    def gather_kernel(x_hbm, inds_hbm, out_hbm, idx_all, *bufs_sems):
        rows = bufs_sems[:_NBUF]
        gs = bufs_sems[_NBUF:2 * _NBUF]
        ws = bufs_sems[2 * _NBUF:]
        wid = lax.axis_index("s") * info.num_cores + lax.axis_index("c")
        base = wid * per_w + jnp.minimum(wid, extra)
        n_my = jnp.where(wid < extra, per_w + 1, per_w)

        pltpu.sync_copy(
            inds_hbm.at[pl.ds(pl.multiple_of(base * _CHUNK, 8), pre)],
            idx_all,
        )

        def start_gather(k, b):
            idx_ref = idx_all.at[pl.ds(pl.multiple_of(k * _CHUNK, 8), _CHUNK)]
            pltpu.async_copy(x_hbm.at[idx_ref], rows[b], gs[b])

        def start_wb(k, b):
            off = pl.multiple_of((base + k) * _CHUNK, 8)
            pltpu.async_copy(rows[b], out_hbm.at[pl.ds(off, _CHUNK)], ws[b])

        def drain_g(b):
            pltpu.make_async_copy(
                x_hbm.at[pl.ds(0, _CHUNK)], rows[b], gs[b]).wait()

        def drain_w(b):
            pltpu.make_async_copy(
                rows[b], out_hbm.at[pl.ds(0, _CHUNK)], ws[b]).wait()

        start_gather(0, 0)

        def group(pk, carry):
            for j in range(1, _NBUF + 1):
                k = _NBUF * pk + j
                b = j % _NBUF

                @pl.when((k >= _NBUF) & (k < n_my))
                def _():
                    drain_w(b)

                @pl.when(k < n_my)
                def _():
                    start_gather(k, b)

                pb = (j - 1) % _NBUF

                @pl.when(k - 1 < n_my)
                def _():
                    drain_g(pb)
                    start_wb(k - 1, pb)

            return carry

        lax.fori_loop(0, (kmax + _NBUF) // _NBUF, group, 0)
        for b in range(_NBUF):
            drain_w(b)

    return gather_kernel(x, inds)


# speedup vs baseline: 2.2130x; 1.0003x over previous
import functools

import jax
import jax.numpy as jnp
from jax import lax
from jax.experimental import pallas as pl
from jax.experimental.pallas import tpu as pltpu
from jax.experimental.pallas import tpu_sc as plsc

_CHUNK = 40
_NBUF = 5


def kernel(x, upsamples):
    n_out = upsamples.shape[0]
    d = x.shape[1]
    info = plsc.get_sparse_core_info()
    nw = info.num_cores * info.num_subcores
    assert n_out % _CHUNK == 0 and _CHUNK % 8 == 0
    n_chunks = n_out // _CHUNK
    per_w = n_chunks // nw
    extra = n_chunks - per_w * nw
    kmax = per_w + (1 if extra else 0)
    pre = kmax * _CHUNK
    pad_to = ((nw - 1) * per_w + min(nw - 1, extra) + kmax) * _CHUNK
    pad_to = max(pad_to, n_out)

    inds = upsamples[:, 0].astype(jnp.int32)
    inds = jnp.pad(inds, (0, pad_to - n_out))
    mesh = plsc.VectorSubcoreMesh(core_axis_name="c", subcore_axis_name="s")

    @functools.partial(
        pl.kernel,
        out_type=jax.ShapeDtypeStruct((n_out, d), jnp.float32),
        mesh=mesh,
        scratch_types=(
            [pltpu.VMEM((pre,), jnp.int32)]
            + [pltpu.VMEM((_CHUNK, d), jnp.float32)] * _NBUF
            + [pltpu.SemaphoreType.DMA] * (2 * _NBUF)
        ),
    )
    def gather_kernel(x_hbm, inds_hbm, out_hbm, idx_all, *bufs_sems):
        rows = bufs_sems[:_NBUF]
        gs = bufs_sems[_NBUF:2 * _NBUF]
        ws = bufs_sems[2 * _NBUF:]
        wid = lax.axis_index("s") * info.num_cores + lax.axis_index("c")
        base = wid * per_w + jnp.minimum(wid, extra)
        n_my = jnp.where(wid < extra, per_w + 1, per_w)

        pltpu.sync_copy(
            inds_hbm.at[pl.ds(pl.multiple_of(base * _CHUNK, 8), pre)],
            idx_all,
        )

        def start_gather(k, b):
            idx_ref = idx_all.at[pl.ds(pl.multiple_of(k * _CHUNK, 8), _CHUNK)]
            pltpu.async_copy(x_hbm.at[idx_ref], rows[b], gs[b])

        def start_wb(k, b):
            off = pl.multiple_of((base + k) * _CHUNK, 8)
            pltpu.async_copy(rows[b], out_hbm.at[pl.ds(off, _CHUNK)], ws[b])

        def drain_g(b):
            pltpu.make_async_copy(
                x_hbm.at[pl.ds(0, _CHUNK)], rows[b], gs[b]).wait()

        def drain_w(b):
            pltpu.make_async_copy(
                rows[b], out_hbm.at[pl.ds(0, _CHUNK)], ws[b]).wait()

        start_gather(0, 0)

        def group(pk, carry):
            for j in range(1, _NBUF + 1):
                k = _NBUF * pk + j
                b = j % _NBUF

                @pl.when((k >= _NBUF) & (k < n_my))
                def _():
                    drain_w(b)

                @pl.when(k < n_my)
                def _():
                    start_gather(k, b)

                pb = (j - 1) % _NBUF

                @pl.when(k - 1 < n_my)
                def _():
                    drain_g(pb)
                    start_wb(k - 1, pb)

            return carry

        lax.fori_loop(0, (kmax + _NBUF) // _NBUF, group, 0)
        for b in range(_NBUF):
            drain_w(b)

    return gather_kernel(x, inds)


# speedup vs baseline: 2.2185x; 1.0025x over previous
import functools

import jax
import jax.numpy as jnp
from jax import lax
from jax.experimental import pallas as pl
from jax.experimental.pallas import tpu as pltpu
from jax.experimental.pallas import tpu_sc as plsc

_CHUNK = 40
_NBUF = 6


def kernel(x, upsamples):
    n_out = upsamples.shape[0]
    d = x.shape[1]
    info = plsc.get_sparse_core_info()
    nw = info.num_cores * info.num_subcores
    assert n_out % _CHUNK == 0 and _CHUNK % 8 == 0
    n_chunks = n_out // _CHUNK
    per_w = n_chunks // nw
    extra = n_chunks - per_w * nw
    kmax = per_w + (1 if extra else 0)
    pre = kmax * _CHUNK
    pad_to = ((nw - 1) * per_w + min(nw - 1, extra) + kmax) * _CHUNK
    pad_to = max(pad_to, n_out)

    inds = upsamples[:, 0].astype(jnp.int32)
    inds = jnp.pad(inds, (0, pad_to - n_out))
    mesh = plsc.VectorSubcoreMesh(core_axis_name="c", subcore_axis_name="s")

    @functools.partial(
        pl.kernel,
        out_type=jax.ShapeDtypeStruct((n_out, d), jnp.float32),
        mesh=mesh,
        scratch_types=(
            [pltpu.VMEM((pre,), jnp.int32)]
            + [pltpu.VMEM((_CHUNK, d), jnp.float32)] * _NBUF
            + [pltpu.SemaphoreType.DMA] * (2 * _NBUF)
        ),
    )
    def gather_kernel(x_hbm, inds_hbm, out_hbm, idx_all, *bufs_sems):
        rows = bufs_sems[:_NBUF]
        gs = bufs_sems[_NBUF:2 * _NBUF]
        ws = bufs_sems[2 * _NBUF:]
        wid = lax.axis_index("s") * info.num_cores + lax.axis_index("c")
        base = wid * per_w + jnp.minimum(wid, extra)
        n_my = jnp.where(wid < extra, per_w + 1, per_w)

        pltpu.sync_copy(
            inds_hbm.at[pl.ds(pl.multiple_of(base * _CHUNK, 8), pre)],
            idx_all,
        )

        def start_gather(k, b):
            idx_ref = idx_all.at[pl.ds(pl.multiple_of(k * _CHUNK, 8), _CHUNK)]
            pltpu.async_copy(x_hbm.at[idx_ref], rows[b], gs[b])

        def start_wb(k, b):
            off = pl.multiple_of((base + k) * _CHUNK, 8)
            pltpu.async_copy(rows[b], out_hbm.at[pl.ds(off, _CHUNK)], ws[b])

        def drain_g(b):
            pltpu.make_async_copy(
                x_hbm.at[pl.ds(0, _CHUNK)], rows[b], gs[b]).wait()

        def drain_w(b):
            pltpu.make_async_copy(
                rows[b], out_hbm.at[pl.ds(0, _CHUNK)], ws[b]).wait()

        start_gather(0, 0)

        def group(pk, carry):
            for j in range(1, _NBUF + 1):
                k = _NBUF * pk + j
                b = j % _NBUF

                @pl.when((k >= _NBUF) & (k < n_my))
                def _():
                    drain_w(b)

                @pl.when(k < n_my)
                def _():
                    start_gather(k, b)

                pb = (j - 1) % _NBUF

                @pl.when(k - 1 < n_my)
                def _():
                    drain_g(pb)
                    start_wb(k - 1, pb)

            return carry

        lax.fori_loop(0, (kmax + _NBUF) // _NBUF, group, 0)
        for b in range(_NBUF):
            drain_w(b)

    return gather_kernel(x, inds)


# speedup vs baseline: 2.2214x; 1.0013x over previous
import functools

import jax
import jax.numpy as jnp
from jax import lax
from jax.experimental import pallas as pl
from jax.experimental.pallas import tpu as pltpu
from jax.experimental.pallas import tpu_sc as plsc

_CHUNK = 40
_NBUF = 6


def kernel(x, upsamples):
    n_out = upsamples.shape[0]
    d = x.shape[1]
    info = plsc.get_sparse_core_info()
    nw = info.num_cores * info.num_subcores
    assert n_out % _CHUNK == 0 and _CHUNK % 8 == 0
    n_chunks = n_out // _CHUNK
    per_w = n_chunks // nw
    extra = n_chunks - per_w * nw
    assert per_w >= _NBUF
    kmax = per_w + (1 if extra else 0)
    pre = kmax * _CHUNK
    pad_to = ((nw - 1) * per_w + min(nw - 1, extra) + kmax) * _CHUNK
    pad_to = max(pad_to, n_out)

    inds = upsamples[:, 0].astype(jnp.int32)
    inds = jnp.pad(inds, (0, pad_to - n_out))
    mesh = plsc.VectorSubcoreMesh(core_axis_name="c", subcore_axis_name="s")

    @functools.partial(
        pl.kernel,
        out_type=jax.ShapeDtypeStruct((n_out, d), jnp.float32),
        mesh=mesh,
        scratch_types=(
            [pltpu.VMEM((pre,), jnp.int32)]
            + [pltpu.VMEM((_CHUNK, d), jnp.float32)] * _NBUF
            + [pltpu.SemaphoreType.DMA] * (2 * _NBUF)
        ),
    )
    def gather_kernel(x_hbm, inds_hbm, out_hbm, idx_all, *bufs_sems):
        rows = bufs_sems[:_NBUF]
        gs = bufs_sems[_NBUF:2 * _NBUF]
        ws = bufs_sems[2 * _NBUF:]
        wid = lax.axis_index("s") * info.num_cores + lax.axis_index("c")
        base = wid * per_w + jnp.minimum(wid, extra)
        n_my = jnp.where(wid < extra, per_w + 1, per_w)

        pltpu.sync_copy(
            inds_hbm.at[pl.ds(pl.multiple_of(base * _CHUNK, 8), pre)],
            idx_all,
        )

        def start_gather(k, b):
            idx_ref = idx_all.at[pl.ds(pl.multiple_of(k * _CHUNK, 8), _CHUNK)]
            pltpu.async_copy(x_hbm.at[idx_ref], rows[b], gs[b])

        def start_wb(k, b):
            off = pl.multiple_of((base + k) * _CHUNK, 8)
            pltpu.async_copy(rows[b], out_hbm.at[pl.ds(off, _CHUNK)], ws[b])

        def drain_g(b):
            pltpu.make_async_copy(
                x_hbm.at[pl.ds(0, _CHUNK)], rows[b], gs[b]).wait()

        def drain_w(b):
            pltpu.make_async_copy(
                rows[b], out_hbm.at[pl.ds(0, _CHUNK)], ws[b]).wait()

        start_gather(0, 0)

        def group(pk, carry):
            for j in range(1, _NBUF + 1):
                k = _NBUF * pk + j
                b = j % _NBUF

                @pl.when((k >= _NBUF) & (k < n_my))
                def _():
                    drain_w(b)

                @pl.when(k < n_my)
                def _():
                    start_gather(k, b)

                pb = (j - 1) % _NBUF

                @pl.when(k - 1 < n_my)
                def _():
                    drain_g(pb)
                    start_wb(k - 1, pb)

            return carry

        lax.fori_loop(0, (kmax + _NBUF) // _NBUF, group, 0)
        for b in range(_NBUF):
            drain_w(b)

    return gather_kernel(x, inds)
